# Initial kernel scaffold; baseline (speedup 1.0000x reference)
import jax
import jax.numpy as jnp
from jax.experimental import pallas as pl


def kernel(x, edge_index, W1, att_src1, att_dst1, bias1, W2, att_src2, att_dst2, bias2):
    raise NotImplementedError("write your pallas kernel here")


# speedup vs baseline: 27.8691x; 27.8691x over previous
import functools

import jax
import jax.numpy as jnp
from jax import lax
from jax.experimental import pallas as pl
from jax.experimental.pallas import tpu as pltpu
from jax.experimental.pallas import tpu_sc as plsc

N = 10000
E = 320000
F_IN = 128
HEADS = 8
HID = 8
NCLS = 40

WREC = 128

NT = 32
KB = 128
NBLK = 80
EPT = NBLK * KB
EPAD = NT * EPT
NACC = 10080

BR = 1000
GRID = N // BR

_f32 = jnp.float32


def _rep_mask(w):
    ri = lax.broadcasted_iota(jnp.int32, (64, 64), 0) // w
    ci = lax.broadcasted_iota(jnp.int32, (64, 64), 1) // w
    return (ri == ci).astype(_f32)


def _prep1_body(x_ref, w1_ref, as_ref, ad_ref, ts_ref, td_ref):
    xp = jnp.dot(x_ref[...], w1_ref[...], preferred_element_type=_f32)
    m = _rep_mask(8)
    a_s = jnp.dot(xp * as_ref[...], m, preferred_element_type=_f32)
    a_d = jnp.dot(xp * ad_ref[...], m, preferred_element_type=_f32)
    ts_ref[...] = jnp.concatenate([xp, a_s], axis=1)
    td_ref[...] = jnp.concatenate([a_d, jnp.zeros((BR, 64), _f32)], axis=1)


def _prep1(x, W1, as1, ad1):
    return pl.pallas_call(
        _prep1_body,
        grid=(GRID,),
        in_specs=[
            pl.BlockSpec((BR, F_IN), lambda i: (i, 0)),
            pl.BlockSpec((F_IN, 64), lambda i: (0, 0)),
            pl.BlockSpec((1, 64), lambda i: (0, 0)),
            pl.BlockSpec((1, 64), lambda i: (0, 0)),
        ],
        out_specs=[
            pl.BlockSpec((BR, WREC), lambda i: (i, 0)),
            pl.BlockSpec((BR, WREC), lambda i: (i, 0)),
        ],
        out_shape=[
            jax.ShapeDtypeStruct((N, WREC), _f32),
            jax.ShapeDtypeStruct((N, WREC), _f32),
        ],
    )(x, W1, as1, ad1)


def _combine1_body(acc_ref, ts_ref, td_ref, b1_ref, w2_ref, as2_ref, ad2_ref,
                   t2s_ref, t2d_ref):
    a = acc_ref[0] + acc_ref[1]
    ts = ts_ref[...]
    xp = ts[:, 0:64]
    s = ts[:, 64:128] + td_ref[...][:, 0:64]
    wsl = jnp.exp(jnp.where(s >= 0., s, 0.2 * s))
    num = a[:, 0:64] + wsl * xp
    den = a[:, 64:128] + wsl
    h1 = jnp.maximum(num / (den + 1e-16) + b1_ref[...], 0.)
    xp2 = jnp.dot(h1, w2_ref[...], preferred_element_type=_f32)
    as2 = jnp.sum(xp2 * as2_ref[...], axis=1, keepdims=True)
    ad2 = jnp.sum(xp2 * ad2_ref[...], axis=1, keepdims=True)
    o16 = jnp.ones((1, 16), _f32)
    z8 = jnp.zeros((BR, 8), _f32)
    z64 = jnp.zeros((BR, 64), _f32)
    t2s_ref[...] = jnp.concatenate([xp2, z8, as2 * o16, z64], axis=1)
    t2d_ref[...] = jnp.concatenate(
        [ad2 * o16, z64, jnp.zeros((BR, 48), _f32)], axis=1)


def _combine1(acc, ts, td, b1, W2, as2, ad2):
    return pl.pallas_call(
        _combine1_body,
        grid=(GRID,),
        in_specs=[
            pl.BlockSpec((2, BR, WREC), lambda i: (0, i, 0)),
            pl.BlockSpec((BR, WREC), lambda i: (i, 0)),
            pl.BlockSpec((BR, WREC), lambda i: (i, 0)),
            pl.BlockSpec((1, 64), lambda i: (0, 0)),
            pl.BlockSpec((64, NCLS), lambda i: (0, 0)),
            pl.BlockSpec((1, NCLS), lambda i: (0, 0)),
            pl.BlockSpec((1, NCLS), lambda i: (0, 0)),
        ],
        out_specs=[
            pl.BlockSpec((BR, WREC), lambda i: (i, 0)),
            pl.BlockSpec((BR, WREC), lambda i: (i, 0)),
        ],
        out_shape=[
            jax.ShapeDtypeStruct((N, WREC), _f32),
            jax.ShapeDtypeStruct((N, WREC), _f32),
        ],
    )(acc, ts, td, b1, W2, as2, ad2)


def _final_body(acc_ref, t2s_ref, t2d_ref, b2_ref, out_ref):
    a = acc_ref[0] + acc_ref[1]
    t = t2s_ref[...]
    s = t[:, 48:49] + t2d_ref[...][:, 0:1]
    w = jnp.exp(jnp.where(s >= 0., s, 0.2 * s))
    num = a[:, 0:40] + w * t[:, 0:40]
    den = a[:, 40:41] + w
    o = num / (den + 1e-16) + b2_ref[...]
    mx = jnp.max(o, axis=1, keepdims=True)
    z = o - mx
    out_ref[...] = z - jnp.log(jnp.sum(jnp.exp(z), axis=1, keepdims=True))


def _final(acc2, t2s, t2d, b2):
    return pl.pallas_call(
        _final_body,
        grid=(GRID,),
        in_specs=[
            pl.BlockSpec((2, BR, WREC), lambda i: (0, i, 0)),
            pl.BlockSpec((BR, WREC), lambda i: (i, 0)),
            pl.BlockSpec((BR, WREC), lambda i: (i, 0)),
            pl.BlockSpec((1, NCLS), lambda i: (0, 0)),
        ],
        out_specs=pl.BlockSpec((BR, NCLS), lambda i: (i, 0)),
        out_shape=jax.ShapeDtypeStruct((N, NCLS), _f32),
    )(acc2, t2s, t2d, b2)


def _sc_edge_kernel(edge_fn):
    mesh = plsc.VectorSubcoreMesh(core_axis_name="c", subcore_axis_name="s")

    @functools.partial(
        pl.kernel,
        mesh=mesh,
        out_type=jax.ShapeDtypeStruct((2, NACC, WREC), _f32),
        scratch_types=[
            pltpu.VMEM((3, KB), jnp.int32),
            pltpu.VMEM((KB, WREC), _f32),
            pltpu.VMEM((KB, WREC), _f32),
            pltpu.VMEM_SHARED((NACC, WREC), _f32),
            pltpu.SemaphoreType.DMA,
            pltpu.SemaphoreType.DMA,
        ],
    )
    def k(tabs, tabd, idx3, out, IDX, S, D, acc, sem_s, sem_d):
        cid = lax.axis_index("c")
        sid = lax.axis_index("s")
        tile = cid * 16 + sid

        def zrow(r, _):
            for c in range(WREC // 16):
                S[r, pl.ds(16 * c, 16)] = jnp.zeros((16,), _f32)
            return 0
        lax.fori_loop(0, KB, zrow, 0)
        for b in range(8):
            if b < 6:
                pltpu.sync_copy(S.at[pl.ds(0, 80)],
                                acc.at[pl.ds(sid * 640 + b * 80, 80)])
            else:
                @pl.when(sid < 15)
                def _():
                    pltpu.sync_copy(S.at[pl.ds(0, 80)],
                                    acc.at[pl.ds(sid * 640 + b * 80, 80)])
        plsc.subcore_barrier()

        def blk(j, _):
            pltpu.sync_copy(idx3.at[tile, j], IDX)
            cs = pltpu.async_copy(tabs.at[IDX.at[0]], S, sem_s)
            cd = pltpu.async_copy(tabd.at[IDX.at[1]], D, sem_d)
            cs.wait()
            cd.wait()

            def edge(e, _):
                edge_fn(S, D, e)
                return 0
            lax.fori_loop(0, KB, edge, 0, unroll=2)
            pltpu.sync_copy(S, acc.at[IDX.at[2]], add=True)
            return 0
        lax.fori_loop(0, NBLK, blk, 0)

        plsc.subcore_barrier()
        for b in range(8):
            if b < 6:
                pltpu.sync_copy(acc.at[pl.ds(sid * 640 + b * 80, 80)],
                                out.at[cid, pl.ds(sid * 640 + b * 80, 80)])
            else:
                @pl.when(sid < 15)
                def _():
                    pltpu.sync_copy(acc.at[pl.ds(sid * 640 + b * 80, 80)],
                                    out.at[cid, pl.ds(sid * 640 + b * 80, 80)])

    return k


def _edge1(S, D, e):
    for h in range(4):
        s = S[e, pl.ds(64 + 16 * h, 16)] + D[e, pl.ds(16 * h, 16)]
        w = jnp.exp(jnp.where(s >= 0., s, 0.2 * s))
        S[e, pl.ds(16 * h, 16)] = S[e, pl.ds(16 * h, 16)] * w
        S[e, pl.ds(64 + 16 * h, 16)] = w


def _edge2(S, D, e):
    iot = lax.broadcasted_iota(jnp.int32, (16,), 0)
    s = S[e, pl.ds(48, 16)] + D[e, pl.ds(0, 16)]
    w = jnp.exp(jnp.where(s >= 0., s, 0.2 * s))
    S[e, pl.ds(0, 16)] = S[e, pl.ds(0, 16)] * w
    S[e, pl.ds(16, 16)] = S[e, pl.ds(16, 16)] * w
    m2 = jnp.where(iot < 8, S[e, pl.ds(32, 16)] * w,
                   jnp.where(iot == 8, w, 0.))
    S[e, pl.ds(32, 16)] = m2


_sc_edges1 = _sc_edge_kernel(_edge1)
_sc_edges2 = _sc_edge_kernel(_edge2)


def kernel(x, edge_index, W1, att_src1, att_dst1, bias1,
           W2, att_src2, att_dst2, bias2):
    ei = edge_index.astype(jnp.int32)
    npad = EPAD - E
    zpad = jnp.zeros((npad,), jnp.int32)
    tpad = N + (jnp.arange(npad, dtype=jnp.int32) % (NACC - N))
    srcg = jnp.concatenate([ei[0], zpad]).reshape(NT, NBLK, KB)
    dstg = jnp.concatenate([ei[1], zpad]).reshape(NT, NBLK, KB)
    dsts = jnp.concatenate([ei[1], tpad]).reshape(NT, NBLK, KB)
    idx3 = jnp.stack([srcg, dstg, dsts], axis=2)
    as1 = att_src1.reshape(1, 64)
    ad1 = att_dst1.reshape(1, 64)
    b1 = bias1.reshape(1, 64)
    b2 = bias2.reshape(1, NCLS)

    ts, td = _prep1(x, W1, as1, ad1)
    acc1 = _sc_edges1(ts, td, idx3)
    t2s, t2d = _combine1(acc1, ts, td, b1, W2, att_src2, att_dst2)
    acc2 = _sc_edges2(t2s, t2d, idx3)
    return _final(acc2, t2s, t2d, b2)


# speedup vs baseline: 40.3405x; 1.4475x over previous
import functools

import jax
import jax.numpy as jnp
from jax import lax
from jax.experimental import pallas as pl
from jax.experimental.pallas import tpu as pltpu
from jax.experimental.pallas import tpu_sc as plsc

N = 10000
E = 320000
F_IN = 128
HEADS = 8
HID = 8
NCLS = 40

WREC = 128

NT = 32
KB = 80
NBLK = 128
NPAIR = NBLK // 2
EPT = NBLK * KB
EPAD = NT * EPT
NACC = 10080

BR = 1000
GRID = N // BR

_f32 = jnp.float32


def _rep_mask(w):
    ri = lax.broadcasted_iota(jnp.int32, (64, 64), 0) // w
    ci = lax.broadcasted_iota(jnp.int32, (64, 64), 1) // w
    return (ri == ci).astype(_f32)


def _prep1_body(x_ref, w1_ref, as_ref, ad_ref, ts_ref, td_ref):
    xp = jnp.dot(x_ref[...], w1_ref[...], preferred_element_type=_f32)
    m = _rep_mask(8)
    a_s = jnp.dot(xp * as_ref[...], m, preferred_element_type=_f32)
    a_d = jnp.dot(xp * ad_ref[...], m, preferred_element_type=_f32)
    ts_ref[...] = jnp.concatenate([xp, a_s], axis=1)
    td_ref[...] = jnp.concatenate([a_d, jnp.zeros((BR, 64), _f32)], axis=1)


def _prep1(x, W1, as1, ad1):
    return pl.pallas_call(
        _prep1_body,
        grid=(GRID,),
        in_specs=[
            pl.BlockSpec((BR, F_IN), lambda i: (i, 0)),
            pl.BlockSpec((F_IN, 64), lambda i: (0, 0)),
            pl.BlockSpec((1, 64), lambda i: (0, 0)),
            pl.BlockSpec((1, 64), lambda i: (0, 0)),
        ],
        out_specs=[
            pl.BlockSpec((BR, WREC), lambda i: (i, 0)),
            pl.BlockSpec((BR, WREC), lambda i: (i, 0)),
        ],
        out_shape=[
            jax.ShapeDtypeStruct((N, WREC), _f32),
            jax.ShapeDtypeStruct((N, WREC), _f32),
        ],
    )(x, W1, as1, ad1)


def _combine1_body(acc_ref, ts_ref, td_ref, b1_ref, w2_ref, as2_ref, ad2_ref,
                   t2s_ref, t2d_ref):
    a = acc_ref[0] + acc_ref[1]
    ts = ts_ref[...]
    xp = ts[:, 0:64]
    s = ts[:, 64:128] + td_ref[...][:, 0:64]
    wsl = jnp.exp(jnp.where(s >= 0., s, 0.2 * s))
    num = a[:, 0:64] + wsl * xp
    den = a[:, 64:128] + wsl
    h1 = jnp.maximum(num / (den + 1e-16) + b1_ref[...], 0.)
    xp2 = jnp.dot(h1, w2_ref[...], preferred_element_type=_f32)
    as2 = jnp.sum(xp2 * as2_ref[...], axis=1, keepdims=True)
    ad2 = jnp.sum(xp2 * ad2_ref[...], axis=1, keepdims=True)
    o16 = jnp.ones((1, 16), _f32)
    z8 = jnp.zeros((BR, 8), _f32)
    z64 = jnp.zeros((BR, 64), _f32)
    t2s_ref[...] = jnp.concatenate([xp2, z8, as2 * o16, z64], axis=1)
    t2d_ref[...] = jnp.concatenate(
        [ad2 * o16, z64, jnp.zeros((BR, 48), _f32)], axis=1)


def _combine1(acc, ts, td, b1, W2, as2, ad2):
    return pl.pallas_call(
        _combine1_body,
        grid=(GRID,),
        in_specs=[
            pl.BlockSpec((2, BR, WREC), lambda i: (0, i, 0)),
            pl.BlockSpec((BR, WREC), lambda i: (i, 0)),
            pl.BlockSpec((BR, WREC), lambda i: (i, 0)),
            pl.BlockSpec((1, 64), lambda i: (0, 0)),
            pl.BlockSpec((64, NCLS), lambda i: (0, 0)),
            pl.BlockSpec((1, NCLS), lambda i: (0, 0)),
            pl.BlockSpec((1, NCLS), lambda i: (0, 0)),
        ],
        out_specs=[
            pl.BlockSpec((BR, WREC), lambda i: (i, 0)),
            pl.BlockSpec((BR, WREC), lambda i: (i, 0)),
        ],
        out_shape=[
            jax.ShapeDtypeStruct((N, WREC), _f32),
            jax.ShapeDtypeStruct((N, WREC), _f32),
        ],
    )(acc, ts, td, b1, W2, as2, ad2)


def _final_body(acc_ref, t2s_ref, t2d_ref, b2_ref, out_ref):
    a = acc_ref[0] + acc_ref[1]
    t = t2s_ref[...]
    s = t[:, 48:49] + t2d_ref[...][:, 0:1]
    w = jnp.exp(jnp.where(s >= 0., s, 0.2 * s))
    num = a[:, 0:40] + w * t[:, 0:40]
    den = a[:, 40:41] + w
    o = num / (den + 1e-16) + b2_ref[...]
    mx = jnp.max(o, axis=1, keepdims=True)
    z = o - mx
    out_ref[...] = z - jnp.log(jnp.sum(jnp.exp(z), axis=1, keepdims=True))


def _final(acc2, t2s, t2d, b2):
    return pl.pallas_call(
        _final_body,
        grid=(GRID,),
        in_specs=[
            pl.BlockSpec((2, BR, WREC), lambda i: (0, i, 0)),
            pl.BlockSpec((BR, WREC), lambda i: (i, 0)),
            pl.BlockSpec((BR, WREC), lambda i: (i, 0)),
            pl.BlockSpec((1, NCLS), lambda i: (0, 0)),
        ],
        out_specs=pl.BlockSpec((BR, NCLS), lambda i: (i, 0)),
        out_shape=jax.ShapeDtypeStruct((N, NCLS), _f32),
    )(acc2, t2s, t2d, b2)


def _sc_edge_kernel(edge_fn):
    mesh = plsc.VectorSubcoreMesh(core_axis_name="c", subcore_axis_name="s")

    @functools.partial(
        pl.kernel,
        mesh=mesh,
        out_type=jax.ShapeDtypeStruct((2, NACC, WREC), _f32),
        scratch_types=[
            pltpu.VMEM((6, KB), jnp.int32),
            pltpu.VMEM((6, KB), jnp.int32),
            pltpu.VMEM((KB, WREC), _f32),
            pltpu.VMEM((KB, WREC), _f32),
            pltpu.VMEM((KB, WREC), _f32),
            pltpu.VMEM((KB, WREC), _f32),
            pltpu.VMEM_SHARED((NACC, WREC), _f32),
            pltpu.SemaphoreType.DMA,
            pltpu.SemaphoreType.DMA,
            pltpu.SemaphoreType.DMA,
            pltpu.SemaphoreType.DMA,
            pltpu.SemaphoreType.DMA,
        ],
    )
    def k(tabs, tabd, idx6, out, P0, P1, SA, DA, SB, DB, acc,
          sem_sa, sem_da, sem_sb, sem_db, sem_i):
        cid = lax.axis_index("c")
        sid = lax.axis_index("s")
        tile = cid * 16 + sid

        def cprows(dst, src, lo, hi):
            for r in range(lo, hi):
                for c in range(KB // 16):
                    dst[r, pl.ds(16 * c, 16)] = src[r, pl.ds(16 * c, 16)]

        def zrow(r, _):
            for c in range(WREC // 16):
                SA[r, pl.ds(16 * c, 16)] = jnp.zeros((16,), _f32)
            return 0
        lax.fori_loop(0, KB, zrow, 0)
        for b in range(8):
            if b < 6:
                pltpu.sync_copy(SA.at[pl.ds(0, 80)],
                                acc.at[pl.ds(sid * 640 + b * 80, 80)])
            else:
                @pl.when(sid < 15)
                def _():
                    pltpu.sync_copy(SA.at[pl.ds(0, 80)],
                                    acc.at[pl.ds(sid * 640 + b * 80, 80)])
        plsc.subcore_barrier()

        pltpu.sync_copy(idx6.at[tile, 0], P0)
        pltpu.async_copy(tabs.at[P0.at[0]], SA, sem_sa)
        pltpu.async_copy(tabd.at[P0.at[1]], DA, sem_da)
        pltpu.async_copy(idx6.at[tile, 1], P1, sem_i)

        def edges(S, D):
            def edge(e, _):
                edge_fn(S, D, e)
                return 0
            lax.fori_loop(0, KB, edge, 0, unroll=4)

        def pair(q, _):
            pltpu.async_copy(tabs.at[P0.at[3]], SB, sem_sb)
            pltpu.async_copy(tabd.at[P0.at[4]], DB, sem_db)
            pltpu.make_async_copy(tabs.at[P0.at[0]], SA, sem_sa).wait()
            pltpu.make_async_copy(tabd.at[P0.at[1]], DA, sem_da).wait()
            edges(SA, DA)
            pltpu.sync_copy(SA, acc.at[P0.at[2]], add=True)
            pltpu.make_async_copy(idx6.at[tile, 0], P1, sem_i).wait()
            cprows(P0, P1, 0, 3)

            @pl.when(q + 1 < NPAIR)
            def _():
                pltpu.async_copy(tabs.at[P0.at[0]], SA, sem_sa)
                pltpu.async_copy(tabd.at[P0.at[1]], DA, sem_da)

            pltpu.make_async_copy(tabs.at[P0.at[3]], SB, sem_sb).wait()
            pltpu.make_async_copy(tabd.at[P0.at[4]], DB, sem_db).wait()
            edges(SB, DB)
            pltpu.sync_copy(SB, acc.at[P0.at[5]], add=True)
            cprows(P0, P1, 3, 6)
            nxt = jnp.minimum(q + 2, NPAIR - 1)
            pltpu.async_copy(idx6.at[tile, nxt], P1, sem_i)
            return 0
        lax.fori_loop(0, NPAIR, pair, 0)

        pltpu.make_async_copy(idx6.at[tile, 0], P1, sem_i).wait()

        plsc.subcore_barrier()
        for b in range(8):
            if b < 6:
                pltpu.sync_copy(acc.at[pl.ds(sid * 640 + b * 80, 80)],
                                out.at[cid, pl.ds(sid * 640 + b * 80, 80)])
            else:
                @pl.when(sid < 15)
                def _():
                    pltpu.sync_copy(acc.at[pl.ds(sid * 640 + b * 80, 80)],
                                    out.at[cid, pl.ds(sid * 640 + b * 80, 80)])

    return k


def _edge1(S, D, e):
    for h in range(4):
        s = S[e, pl.ds(64 + 16 * h, 16)] + D[e, pl.ds(16 * h, 16)]
        w = jnp.exp(jnp.where(s >= 0., s, 0.2 * s))
        S[e, pl.ds(16 * h, 16)] = S[e, pl.ds(16 * h, 16)] * w
        S[e, pl.ds(64 + 16 * h, 16)] = w


def _edge2(S, D, e):
    iot = lax.broadcasted_iota(jnp.int32, (16,), 0)
    s = S[e, pl.ds(48, 16)] + D[e, pl.ds(0, 16)]
    w = jnp.exp(jnp.where(s >= 0., s, 0.2 * s))
    S[e, pl.ds(0, 16)] = S[e, pl.ds(0, 16)] * w
    S[e, pl.ds(16, 16)] = S[e, pl.ds(16, 16)] * w
    m2 = jnp.where(iot < 8, S[e, pl.ds(32, 16)] * w,
                   jnp.where(iot == 8, w, 0.))
    S[e, pl.ds(32, 16)] = m2


_sc_edges1 = _sc_edge_kernel(_edge1)
_sc_edges2 = _sc_edge_kernel(_edge2)


def kernel(x, edge_index, W1, att_src1, att_dst1, bias1,
           W2, att_src2, att_dst2, bias2):
    ei = edge_index.astype(jnp.int32)
    npad = EPAD - E
    zpad = jnp.zeros((npad,), jnp.int32)
    tpad = N + (jnp.arange(npad, dtype=jnp.int32) % (NACC - N))
    srcg = jnp.concatenate([ei[0], zpad]).reshape(NT, NBLK, KB)
    dstg = jnp.concatenate([ei[1], zpad]).reshape(NT, NBLK, KB)
    dsts = jnp.concatenate([ei[1], tpad]).reshape(NT, NBLK, KB)
    idx6 = jnp.stack([srcg[:, 0::2], dstg[:, 0::2], dsts[:, 0::2],
                      srcg[:, 1::2], dstg[:, 1::2], dsts[:, 1::2]], axis=2)
    as1 = att_src1.reshape(1, 64)
    ad1 = att_dst1.reshape(1, 64)
    b1 = bias1.reshape(1, 64)
    b2 = bias2.reshape(1, NCLS)

    ts, td = _prep1(x, W1, as1, ad1)
    acc1 = _sc_edges1(ts, td, idx6)
    t2s, t2d = _combine1(acc1, ts, td, b1, W2, att_src2, att_dst2)
    acc2 = _sc_edges2(t2s, t2d, idx6)
    return _final(acc2, t2s, t2d, b2)
